# baseline (speedup 1.0000x reference)
import functools
import jax
import jax.numpy as jnp
from jax import lax
from jax.experimental import pallas as pl
from jax.experimental.pallas import tpu as pltpu
from jax.experimental.pallas import tpu_sc as plsc

_B = 16384
_F = 26
_VOCAB = 100000
_DIM = 32
_LANES = 16
_NC = 2
_NS = 16
_NW = _NC * _NS
_BPW = _B // _NW
def _relayout_body(src_ref, out_ref, s0, s1, sem0, sem1):
    f = pl.program_id(0)
    r = lax.broadcasted_iota(jnp.int32, (_DIM, _DIM), 0)
    c = lax.broadcasted_iota(jnp.int32, (_DIM, _DIM), 1)
    eye = (r == c).astype(jnp.float32)
    scr = (s0, s1)
    sems = (sem0, sem1)
    chunks = [(k * 3200, 3200) for k in range(31)] + [(99200, 800)]
    dmas = [None, None]

    for i, (off, sz) in enumerate(chunks):
        p = i % 2
        if dmas[p] is not None:
            dmas[p].wait()
        tc = lax.dot_general(
            src_ref[0, :, pl.ds(off, sz)], eye, (((0,), (0,)), ((), ())),
            preferred_element_type=jnp.float32)
        scr[p][pl.ds(0, sz), :] = tc
        d = pltpu.make_async_copy(
            scr[p].at[pl.ds(0, sz), :],
            out_ref.at[pl.ds(f * _VOCAB + off, sz), :],
            sems[p])
        d.start()
        dmas[p] = d

    dmas[0].wait()
    dmas[1].wait()


def _relayout(tabs_t):
    return pl.pallas_call(
        _relayout_body,
        grid=(_F,),
        in_specs=[pl.BlockSpec((1, _DIM, _VOCAB), lambda f: (f, 0, 0))],
        out_specs=pl.BlockSpec(memory_space=pl.ANY),
        out_shape=jax.ShapeDtypeStruct((_F * _VOCAB, _DIM), jnp.float32),
        scratch_shapes=[
            pltpu.VMEM((3200, _DIM), jnp.float32),
            pltpu.VMEM((3200, _DIM), jnp.float32),
            pltpu.SemaphoreType.DMA,
            pltpu.SemaphoreType.DMA,
        ],
    )(tabs_t)


def _sc_body(idx_hbm, tab_hbm, out_hbm,
             idx_v, buf0, buf1, acc_v, sem0, sem1):
    wid = lax.axis_index("s") * _NC + lax.axis_index("c")
    base = wid * _BPW

    pltpu.sync_copy(idx_hbm.at[:, pl.ds(base, _BPW)], idx_v)

    bufs = (buf0, buf1)
    sems = (sem0, sem1)

    def fire(f):
        p = f % 2
        return pltpu.async_copy(tab_hbm.at[idx_v.at[f]], bufs[p], sems[p])

    def accum(f, copy_desc):
        copy_desc.wait()
        buf = bufs[f % 2]
        if f == 0:
            @pl.loop(0, _BPW, unroll=8)
            def _(j):
                acc_v[j, pl.ds(0, _LANES)] = buf[j, pl.ds(0, _LANES)]
                acc_v[j, pl.ds(_LANES, _LANES)] = buf[j, pl.ds(_LANES, _LANES)]
        else:
            @pl.loop(0, _BPW, unroll=8)
            def _(j):
                plsc.addupdate(acc_v.at[j, pl.ds(0, _LANES)],
                               buf[j, pl.ds(0, _LANES)])
                plsc.addupdate(acc_v.at[j, pl.ds(_LANES, _LANES)],
                               buf[j, pl.ds(_LANES, _LANES)])

    d = {0: fire(0), 1: fire(1)}
    for f in range(_F):
        accum(f, d[f % 2])
        nf = f + 2
        if nf < _F:
            d[nf % 2] = fire(nf)

    pltpu.sync_copy(acc_v, out_hbm.at[pl.ds(base, _BPW)])


@jax.jit
def kernel(inputs, tables):
    off = jnp.arange(_F, dtype=jnp.int32) * _VOCAB
    gidx = jnp.asarray(inputs, jnp.int32).T + off[:, None]
    tabs_t = tables.transpose(0, 2, 1)
    flat = _relayout(tabs_t)
    mesh = plsc.VectorSubcoreMesh(core_axis_name="c", subcore_axis_name="s")
    run = pl.kernel(
        _sc_body,
        out_type=jax.ShapeDtypeStruct((_B, _DIM), jnp.float32),
        mesh=mesh,
        compiler_params=pltpu.CompilerParams(use_tc_tiling_on_sc=False),
        scratch_types=[
            pltpu.VMEM((_F, _BPW), jnp.int32),
            pltpu.VMEM((_BPW, _DIM), jnp.float32),
            pltpu.VMEM((_BPW, _DIM), jnp.float32),
            pltpu.VMEM((_BPW, _DIM), jnp.float32),
            pltpu.SemaphoreType.DMA,
            pltpu.SemaphoreType.DMA,
        ],
    )
    return run(gidx, flat)


# speedup vs baseline: 1.4594x; 1.4594x over previous
import functools
import jax
import jax.numpy as jnp
from jax import lax
from jax.experimental import pallas as pl
from jax.experimental.pallas import tpu as pltpu
from jax.experimental.pallas import tpu_sc as plsc

_B = 16384
_F = 26
_VOCAB = 100000
_DIM = 32
_LANES = 16
_NC = 2
_NS = 16
_NW = _NC * _NS
_BPW = _B // _NW
def _relayout_body(src_ref, out_ref):
    r = lax.broadcasted_iota(jnp.int32, (_DIM, _DIM), 0)
    c = lax.broadcasted_iota(jnp.int32, (_DIM, _DIM), 1)
    eye = (r == c).astype(jnp.float32)

    def chunk(off, sz):
        tc = lax.dot_general(
            src_ref[0, :, pl.ds(off, sz)], eye, (((0,), (0,)), ((), ())),
            preferred_element_type=jnp.float32)
        t4 = tc.reshape(sz // 4, 4, _DIM)
        packed = jnp.concatenate([t4[:, k, :] for k in range(4)], axis=1)
        out_ref[0, pl.ds(off // 4, sz // 4), :] = packed

    def body(k, carry):
        chunk(k * 4608, 2304)
        chunk(k * 4608 + 2304, 2304)
        return carry

    lax.fori_loop(0, 21, body, 0)
    chunk(96768, 3232)


def _relayout(tabs_t):
    out = pl.pallas_call(
        _relayout_body,
        grid=(_F,),
        in_specs=[pl.BlockSpec((1, _DIM, _VOCAB), lambda f: (f, 0, 0))],
        out_specs=pl.BlockSpec((1, _VOCAB // 4, 4 * _DIM), lambda f: (f, 0, 0)),
        out_shape=jax.ShapeDtypeStruct((_F, _VOCAB // 4, 4 * _DIM), jnp.float32),
    )(tabs_t)
    return out.reshape(_F * _VOCAB, _DIM)


def _sc_body(idx_hbm, tab_hbm, out_hbm,
             idx_v, buf0, buf1, acc_v, sem0, sem1):
    wid = lax.axis_index("s") * _NC + lax.axis_index("c")
    base = wid * _BPW

    pltpu.sync_copy(idx_hbm.at[:, pl.ds(base, _BPW)], idx_v)

    bufs = (buf0, buf1)
    sems = (sem0, sem1)

    def fire(f):
        p = f % 2
        return pltpu.async_copy(tab_hbm.at[idx_v.at[f]], bufs[p], sems[p])

    def accum(f, copy_desc):
        copy_desc.wait()
        buf = bufs[f % 2]
        if f == 0:
            @pl.loop(0, _BPW, unroll=8)
            def _(j):
                acc_v[j, pl.ds(0, _LANES)] = buf[j, pl.ds(0, _LANES)]
                acc_v[j, pl.ds(_LANES, _LANES)] = buf[j, pl.ds(_LANES, _LANES)]
        else:
            @pl.loop(0, _BPW, unroll=8)
            def _(j):
                plsc.addupdate(acc_v.at[j, pl.ds(0, _LANES)],
                               buf[j, pl.ds(0, _LANES)])
                plsc.addupdate(acc_v.at[j, pl.ds(_LANES, _LANES)],
                               buf[j, pl.ds(_LANES, _LANES)])

    d = {0: fire(0), 1: fire(1)}
    for f in range(_F):
        accum(f, d[f % 2])
        nf = f + 2
        if nf < _F:
            d[nf % 2] = fire(nf)

    pltpu.sync_copy(acc_v, out_hbm.at[pl.ds(base, _BPW)])


@jax.jit
def kernel(inputs, tables):
    off = jnp.arange(_F, dtype=jnp.int32) * _VOCAB
    gidx = jnp.asarray(inputs, jnp.int32).T + off[:, None]
    tabs_t = tables.transpose(0, 2, 1)
    flat = _relayout(tabs_t)
    mesh = plsc.VectorSubcoreMesh(core_axis_name="c", subcore_axis_name="s")
    run = pl.kernel(
        _sc_body,
        out_type=jax.ShapeDtypeStruct((_B, _DIM), jnp.float32),
        mesh=mesh,
        compiler_params=pltpu.CompilerParams(use_tc_tiling_on_sc=False),
        scratch_types=[
            pltpu.VMEM((_F, _BPW), jnp.int32),
            pltpu.VMEM((_BPW, _DIM), jnp.float32),
            pltpu.VMEM((_BPW, _DIM), jnp.float32),
            pltpu.VMEM((_BPW, _DIM), jnp.float32),
            pltpu.SemaphoreType.DMA,
            pltpu.SemaphoreType.DMA,
        ],
    )
    return run(gidx, flat)
